# baseline (speedup 1.0000x reference)
import jax
import jax.numpy as jnp
from jax import lax
from jax.experimental import pallas as pl
from jax.experimental.pallas import tpu as pltpu
from jax.experimental.pallas import tpu_sc as plsc

D_MODEL = 1024
D_EXPERT = 2048
N_EXPERTS = 8
EPS = 1e-6
LANES = 128
TB = 1024
M_BLK = 512
N_BLK = 24
BUF_ROWS = N_BLK * M_BLK
H_CHUNK = 512
N_H = D_EXPERT // H_CHUNK
NC, NS = 2, 16
N_WORKERS = NC * NS
SC_CHUNK = 128
MASK_HI = -65536


def _pack_rows(f32_rows):
    half = f32_rows.shape[1] // 2
    bits = lax.bitcast_convert_type(f32_rows, jnp.int32) + 32768
    lo = lax.shift_right_logical(bits[:, :half], 16)
    hi = jnp.bitwise_and(bits[:, half:], MASK_HI)
    return jnp.bitwise_or(hi, lo)


def _unpack_rows(i32_rows):
    lo = lax.bitcast_convert_type(
        lax.shift_left(i32_rows, 16), jnp.float32)
    hi = lax.bitcast_convert_type(
        jnp.bitwise_and(i32_rows, MASK_HI), jnp.float32)
    return jnp.concatenate([lo, hi], axis=1)
D_HALF = D_MODEL // 2


def _stage_a(x_ref, nw_ref, rw_ref, ltri_ref, xn_ref, meta_ref, cnt_ref,
             carry_ref):
    i = pl.program_id(0)

    @pl.when(i == 0)
    def _init():
        carry_ref[...] = jnp.zeros_like(carry_ref)

    xb = x_ref[...]
    var = jnp.mean(xb * xb, axis=1, keepdims=True)
    xn = xb * lax.rsqrt(var + EPS) * nw_ref[...]
    xn_ref[...] = _pack_rows(xn)
    logits = jnp.dot(xn, rw_ref[...], preferred_element_type=jnp.float32)
    lane = jax.lax.broadcasted_iota(jnp.int32, logits.shape, 1)
    valid = lane < N_EXPERTS
    masked = jnp.where(valid, logits, -1e30)
    mx = jnp.max(masked, axis=1, keepdims=True)
    ex = jnp.where(valid, jnp.exp(masked - mx), 0.0)
    probs = ex / jnp.sum(ex, axis=1, keepdims=True)
    m1 = jnp.max(probs, axis=1, keepdims=True)
    second = jnp.max(jnp.where(probs == m1, -1.0, probs), axis=1,
                     keepdims=True)
    keep = (probs == m1) | (probs == second)
    mf = jnp.where(keep, 1.0, 0.0)
    mcum = jnp.dot(ltri_ref[...], mf.astype(jnp.bfloat16),
                   preferred_element_type=jnp.float32) + carry_ref[...]
    lanef = lane.astype(jnp.float32)
    e0 = jnp.min(jnp.where(keep, lanef, 1e9), axis=1, keepdims=True)
    e1 = jnp.max(jnp.where(keep, lanef, -1.0), axis=1, keepdims=True)
    oh0 = lanef == e0
    oh1 = lanef == e1
    p0 = jnp.sum(jnp.where(oh0, probs, 0.0), axis=1, keepdims=True)
    p1 = jnp.sum(jnp.where(oh1, probs, 0.0), axis=1, keepdims=True)
    r0 = jnp.sum(jnp.where(oh0, mcum, 0.0), axis=1, keepdims=True)
    r1 = jnp.sum(jnp.where(oh1, mcum, 0.0), axis=1, keepdims=True)
    new_carry = carry_ref[...] + jnp.sum(mf, axis=0, keepdims=True)
    carry_ref[...] = new_carry
    cnt_ref[...] = new_carry
    meta = (jnp.where(lane == 0, r0, 0.0) + jnp.where(lane == 1, r1, 0.0)
            + jnp.where(lane == 2, e0, 0.0) + jnp.where(lane == 3, e1, 0.0)
            + jnp.where(lane == 4, p0, 0.0) + jnp.where(lane == 5, p1, 0.0))
    meta_ref[...] = meta


def _stage_a3(cnt_ref, meta_ref, pos0_ref, pos1_ref, bexp_ref, bnr_ref):
    cnt = cnt_ref[...]
    padded = jnp.floor((cnt + (M_BLK - 1)) * (1.0 / M_BLK)) * M_BLK
    rr = jax.lax.broadcasted_iota(jnp.int32, (LANES, LANES), 0)
    cc = jax.lax.broadcasted_iota(jnp.int32, (LANES, LANES), 1)
    ut = jnp.where(rr < cc, 1.0, 0.0)
    base = jnp.dot(padded, ut, preferred_element_type=jnp.float32)
    lane1 = jax.lax.broadcasted_iota(jnp.int32, (1, LANES), 1)

    meta = meta_ref[...]
    lane = jax.lax.broadcasted_iota(jnp.int32, meta.shape, 1)
    r0 = jnp.sum(jnp.where(lane == 0, meta, 0.0), axis=1, keepdims=True)
    r1 = jnp.sum(jnp.where(lane == 1, meta, 0.0), axis=1, keepdims=True)
    e0 = jnp.sum(jnp.where(lane == 2, meta, 0.0), axis=1, keepdims=True)
    e1 = jnp.sum(jnp.where(lane == 3, meta, 0.0), axis=1, keepdims=True)
    pos0 = r0
    pos1 = r1
    bexp = jnp.full((1, LANES), -1.0)
    bnr = jnp.zeros((1, LANES))
    bidx = lane1.astype(jnp.float32)
    for e in range(N_EXPERTS):
        one_e = jnp.where(lane1 == e, 1.0, 0.0)
        base_e = jnp.sum(one_e * base)
        cnt_e = jnp.sum(one_e * cnt)
        bs_e = base_e * (1.0 / M_BLK)
        pos0 = pos0 + jnp.where(e0 == e, base_e, 0.0)
        pos1 = pos1 + jnp.where(e1 == e, base_e, 0.0)
        bexp = bexp + jnp.where(bidx >= bs_e, 1.0, 0.0)
        nr_e = jnp.clip(cnt_e - (bidx - bs_e) * M_BLK, 0.0, float(M_BLK))
        bnr = bnr + jnp.where(bidx >= bs_e, nr_e - bnr, 0.0)
    pos0_ref[...] = pos0.astype(jnp.int32)
    pos1_ref[...] = pos1.astype(jnp.int32)
    bexp_ref[...] = jnp.clip(bexp, 0.0, N_EXPERTS - 1.0).astype(jnp.int32)
    bnr_ref[...] = bnr.astype(jnp.int32)


def _sc_scatter(xn, pos0, pos1):
    mesh = plsc.VectorSubcoreMesh(core_axis_name="c", subcore_axis_name="s")
    n_tok = xn.shape[0]
    per_w = n_tok // N_WORKERS

    def body(xn_hbm, p0_hbm, p1_hbm, buf_hbm, idx0_v, idx1_v, rows_v,
             sem0, sem1):
        wid = lax.axis_index("s") * NC + lax.axis_index("c")
        for k in range(per_w // SC_CHUNK):
            start = wid * per_w + k * SC_CHUNK
            pltpu.sync_copy(p0_hbm.at[pl.ds(start, SC_CHUNK)], idx0_v)
            pltpu.sync_copy(p1_hbm.at[pl.ds(start, SC_CHUNK)], idx1_v)
            pltpu.sync_copy(xn_hbm.at[pl.ds(start, SC_CHUNK)], rows_v)
            c0 = pltpu.async_copy(rows_v, buf_hbm.at[idx0_v], sem0)
            c1 = pltpu.async_copy(rows_v, buf_hbm.at[idx1_v], sem1)
            c0.wait()
            c1.wait()

    return pl.kernel(
        body,
        out_type=jax.ShapeDtypeStruct((BUF_ROWS, D_HALF), jnp.int32),
        mesh=mesh,
        scratch_types=[
            pltpu.VMEM((SC_CHUNK,), jnp.int32),
            pltpu.VMEM((SC_CHUNK,), jnp.int32),
            pltpu.VMEM((SC_CHUNK, D_HALF), jnp.int32),
            pltpu.SemaphoreType.DMA,
            pltpu.SemaphoreType.DMA,
        ],
    )(xn, pos0, pos1)


def _sc_gather(y, pos0, pos1):
    mesh = plsc.VectorSubcoreMesh(core_axis_name="c", subcore_axis_name="s")
    n_tok = pos0.shape[0]
    per_w = n_tok // N_WORKERS

    def body(y_hbm, p0_hbm, p1_hbm, y0_hbm, y1_hbm, idx_v, rows_v, sem):
        wid = lax.axis_index("s") * NC + lax.axis_index("c")
        for k in range(per_w // SC_CHUNK):
            start = wid * per_w + k * SC_CHUNK
            pltpu.sync_copy(p0_hbm.at[pl.ds(start, SC_CHUNK)], idx_v)
            pltpu.async_copy(y_hbm.at[idx_v], rows_v, sem).wait()
            pltpu.sync_copy(rows_v, y0_hbm.at[pl.ds(start, SC_CHUNK)])
            pltpu.sync_copy(p1_hbm.at[pl.ds(start, SC_CHUNK)], idx_v)
            pltpu.async_copy(y_hbm.at[idx_v], rows_v, sem).wait()
            pltpu.sync_copy(rows_v, y1_hbm.at[pl.ds(start, SC_CHUNK)])

    return pl.kernel(
        body,
        out_type=[
            jax.ShapeDtypeStruct((n_tok, D_HALF), jnp.int32),
            jax.ShapeDtypeStruct((n_tok, D_HALF), jnp.int32),
        ],
        mesh=mesh,
        scratch_types=[
            pltpu.VMEM((SC_CHUNK,), jnp.int32),
            pltpu.VMEM((SC_CHUNK, D_HALF), jnp.int32),
            pltpu.SemaphoreType.DMA,
        ],
    )(y, pos0, pos1)


def _stage_b(bexp_ref, bnr_ref, buf_ref, wg_ref, wu_ref, wd_ref, y_ref,
             acc_ref):
    b = pl.program_id(0)
    h = pl.program_id(1)

    @pl.when(bnr_ref[b] > 0)
    def _compute():
        xb = _unpack_rows(buf_ref[...]).astype(jnp.bfloat16)
        g = jnp.dot(xb, wg_ref[0], preferred_element_type=jnp.float32)
        u = jnp.dot(xb, wu_ref[0], preferred_element_type=jnp.float32)
        hid = (g * jax.lax.logistic(g) * u).astype(jnp.bfloat16)
        d = jnp.dot(hid, wd_ref[0], preferred_element_type=jnp.float32)

        @pl.when(h == 0)
        def _first():
            acc_ref[...] = d

        @pl.when(h > 0)
        def _rest():
            acc_ref[...] += d

        @pl.when(h == N_H - 1)
        def _flush():
            y_ref[...] = _pack_rows(acc_ref[...])


def _stage_c(x_ref, y0_ref, y1_ref, meta_ref, out_ref):
    meta = meta_ref[...]
    lane = jax.lax.broadcasted_iota(jnp.int32, meta.shape, 1)
    p0 = jnp.sum(jnp.where(lane == 4, meta, 0.0), axis=1, keepdims=True)
    p1 = jnp.sum(jnp.where(lane == 5, meta, 0.0), axis=1, keepdims=True)
    y0 = _unpack_rows(y0_ref[...])
    y1 = _unpack_rows(y1_ref[...])
    out_ref[...] = x_ref[...] + p0 * y0 + p1 * y1


def kernel(x, norm_w, router_w, Wg, Wu, Wd):
    b, s, d = x.shape
    n_tok = b * s
    x_flat = x.reshape(n_tok, d)
    nw = norm_w.reshape(1, d)
    rw_pad = jnp.zeros((d, LANES), jnp.float32).at[:, :N_EXPERTS].set(router_w)
    rr = jax.lax.broadcasted_iota(jnp.int32, (TB, TB), 0)
    cc = jax.lax.broadcasted_iota(jnp.int32, (TB, TB), 1)
    ltri = jnp.where(rr > cc, 1.0, 0.0).astype(jnp.bfloat16)

    n_tb = n_tok // TB
    xn, meta, cnt = pl.pallas_call(
        _stage_a,
        grid=(n_tb,),
        in_specs=[
            pl.BlockSpec((TB, d), lambda i: (i, 0)),
            pl.BlockSpec((1, d), lambda i: (0, 0)),
            pl.BlockSpec((d, LANES), lambda i: (0, 0)),
            pl.BlockSpec((TB, TB), lambda i: (0, 0)),
        ],
        out_specs=[
            pl.BlockSpec((TB, D_HALF), lambda i: (i, 0)),
            pl.BlockSpec((TB, LANES), lambda i: (i, 0)),
            pl.BlockSpec((1, LANES), lambda i: (0, 0)),
        ],
        out_shape=[
            jax.ShapeDtypeStruct((n_tok, D_HALF), jnp.int32),
            jax.ShapeDtypeStruct((n_tok, LANES), jnp.float32),
            jax.ShapeDtypeStruct((1, LANES), jnp.float32),
        ],
        scratch_shapes=[pltpu.VMEM((1, LANES), jnp.float32)],
        compiler_params=pltpu.CompilerParams(
            dimension_semantics=("arbitrary",)),
    )(x_flat, nw, rw_pad, ltri)

    pos0, pos1, bexp, bnr = pl.pallas_call(
        _stage_a3,
        grid=(n_tb,),
        in_specs=[
            pl.BlockSpec((1, LANES), lambda i: (0, 0)),
            pl.BlockSpec((TB, LANES), lambda i: (i, 0)),
        ],
        out_specs=[
            pl.BlockSpec((TB, 1), lambda i: (i, 0)),
            pl.BlockSpec((TB, 1), lambda i: (i, 0)),
            pl.BlockSpec((1, LANES), lambda i: (0, 0)),
            pl.BlockSpec((1, LANES), lambda i: (0, 0)),
        ],
        out_shape=[
            jax.ShapeDtypeStruct((n_tok, 1), jnp.int32),
            jax.ShapeDtypeStruct((n_tok, 1), jnp.int32),
            jax.ShapeDtypeStruct((1, LANES), jnp.int32),
            jax.ShapeDtypeStruct((1, LANES), jnp.int32),
        ],
        compiler_params=pltpu.CompilerParams(
            dimension_semantics=("arbitrary",)),
    )(cnt, meta)

    pos0_f = pos0.reshape(n_tok)
    pos1_f = pos1.reshape(n_tok)
    buf = _sc_scatter(xn, pos0_f, pos1_f)

    bexp_s = bexp.reshape(LANES)[:N_BLK]
    bnr_s = bnr.reshape(LANES)[:N_BLK]
    wg16 = Wg.astype(jnp.bfloat16)
    wu16 = Wu.astype(jnp.bfloat16)
    wd16 = Wd.astype(jnp.bfloat16)

    y = pl.pallas_call(
        _stage_b,
        grid_spec=pltpu.PrefetchScalarGridSpec(
            num_scalar_prefetch=2,
            grid=(N_BLK, N_H),
            in_specs=[
                pl.BlockSpec((M_BLK, D_HALF), lambda b, h, be, bn: (b, 0)),
                pl.BlockSpec((1, d, H_CHUNK),
                             lambda b, h, be, bn: (be[b], 0, h)),
                pl.BlockSpec((1, d, H_CHUNK),
                             lambda b, h, be, bn: (be[b], 0, h)),
                pl.BlockSpec((1, H_CHUNK, d),
                             lambda b, h, be, bn: (be[b], h, 0)),
            ],
            out_specs=pl.BlockSpec((M_BLK, D_HALF),
                                   lambda b, h, be, bn: (b, 0)),
            scratch_shapes=[pltpu.VMEM((M_BLK, D_MODEL), jnp.float32)],
        ),
        out_shape=jax.ShapeDtypeStruct((BUF_ROWS, D_HALF), jnp.int32),
        compiler_params=pltpu.CompilerParams(
            dimension_semantics=("arbitrary", "arbitrary")),
    )(bexp_s, bnr_s, buf, wg16, wu16, wd16)

    y0, y1 = _sc_gather(y, pos0_f, pos1_f)

    out = pl.pallas_call(
        _stage_c,
        grid=(n_tb,),
        in_specs=[
            pl.BlockSpec((TB, d), lambda i: (i, 0)),
            pl.BlockSpec((TB, D_HALF), lambda i: (i, 0)),
            pl.BlockSpec((TB, D_HALF), lambda i: (i, 0)),
            pl.BlockSpec((TB, LANES), lambda i: (i, 0)),
        ],
        out_specs=pl.BlockSpec((TB, d), lambda i: (i, 0)),
        out_shape=jax.ShapeDtypeStruct((n_tok, d), jnp.float32),
    )(x_flat, y0, y1, meta)

    return out.reshape(b, s, d)


# speedup vs baseline: 1.1720x; 1.1720x over previous
import jax
import jax.numpy as jnp
from jax import lax
from jax.experimental import pallas as pl
from jax.experimental.pallas import tpu as pltpu
from jax.experimental.pallas import tpu_sc as plsc

D_MODEL = 1024
D_EXPERT = 2048
N_EXPERTS = 8
EPS = 1e-6
LANES = 128
TB = 1024
M_BLK = 512
N_BLK = 24
BUF_ROWS = N_BLK * M_BLK
H_CHUNK = 512
N_H = D_EXPERT // H_CHUNK
NC, NS = 2, 16
N_WORKERS = NC * NS
SC_CHUNK = 128
MASK_HI = -65536


def _pack_rows(f32_rows):
    half = f32_rows.shape[1] // 2
    bits = lax.bitcast_convert_type(f32_rows, jnp.int32) + 32768
    lo = lax.shift_right_logical(bits[:, :half], 16)
    hi = jnp.bitwise_and(bits[:, half:], MASK_HI)
    return jnp.bitwise_or(hi, lo)


def _unpack_rows(i32_rows):
    lo = lax.bitcast_convert_type(
        lax.shift_left(i32_rows, 16), jnp.float32)
    hi = lax.bitcast_convert_type(
        jnp.bitwise_and(i32_rows, MASK_HI), jnp.float32)
    return jnp.concatenate([lo, hi], axis=1)
D_HALF = D_MODEL // 2


def _stage_a(x_ref, nw_ref, rw_ref, ltri_ref, xn_ref, meta_ref, cnt_ref,
             carry_ref):
    i = pl.program_id(0)

    @pl.when(i == 0)
    def _init():
        carry_ref[...] = jnp.zeros_like(carry_ref)

    xb = x_ref[...]
    var = jnp.mean(xb * xb, axis=1, keepdims=True)
    xn = xb * lax.rsqrt(var + EPS) * nw_ref[...]
    xn_ref[...] = _pack_rows(xn)
    logits = jnp.dot(xn, rw_ref[...], preferred_element_type=jnp.float32)
    lane = jax.lax.broadcasted_iota(jnp.int32, logits.shape, 1)
    valid = lane < N_EXPERTS
    masked = jnp.where(valid, logits, -1e30)
    mx = jnp.max(masked, axis=1, keepdims=True)
    ex = jnp.where(valid, jnp.exp(masked - mx), 0.0)
    probs = ex / jnp.sum(ex, axis=1, keepdims=True)
    m1 = jnp.max(probs, axis=1, keepdims=True)
    second = jnp.max(jnp.where(probs == m1, -1.0, probs), axis=1,
                     keepdims=True)
    keep = (probs == m1) | (probs == second)
    mf = jnp.where(keep, 1.0, 0.0)
    mcum = jnp.dot(ltri_ref[...], mf.astype(jnp.bfloat16),
                   preferred_element_type=jnp.float32) + carry_ref[...]
    lanef = lane.astype(jnp.float32)
    e0 = jnp.min(jnp.where(keep, lanef, 1e9), axis=1, keepdims=True)
    e1 = jnp.max(jnp.where(keep, lanef, -1.0), axis=1, keepdims=True)
    oh0 = lanef == e0
    oh1 = lanef == e1
    p0 = jnp.sum(jnp.where(oh0, probs, 0.0), axis=1, keepdims=True)
    p1 = jnp.sum(jnp.where(oh1, probs, 0.0), axis=1, keepdims=True)
    r0 = jnp.sum(jnp.where(oh0, mcum, 0.0), axis=1, keepdims=True)
    r1 = jnp.sum(jnp.where(oh1, mcum, 0.0), axis=1, keepdims=True)
    new_carry = carry_ref[...] + jnp.sum(mf, axis=0, keepdims=True)
    carry_ref[...] = new_carry
    cnt_ref[...] = new_carry
    meta = (jnp.where(lane == 0, r0, 0.0) + jnp.where(lane == 1, r1, 0.0)
            + jnp.where(lane == 2, e0, 0.0) + jnp.where(lane == 3, e1, 0.0)
            + jnp.where(lane == 4, p0, 0.0) + jnp.where(lane == 5, p1, 0.0))
    meta_ref[...] = meta


def _stage_a3(cnt_ref, meta_ref, pos0_ref, pos1_ref, bexp_ref, bnr_ref):
    cnt = cnt_ref[...]
    padded = jnp.floor((cnt + (M_BLK - 1)) * (1.0 / M_BLK)) * M_BLK
    rr = jax.lax.broadcasted_iota(jnp.int32, (LANES, LANES), 0)
    cc = jax.lax.broadcasted_iota(jnp.int32, (LANES, LANES), 1)
    ut = jnp.where(rr < cc, 1.0, 0.0)
    base = jnp.dot(padded, ut, preferred_element_type=jnp.float32)
    lane1 = jax.lax.broadcasted_iota(jnp.int32, (1, LANES), 1)

    meta = meta_ref[...]
    lane = jax.lax.broadcasted_iota(jnp.int32, meta.shape, 1)
    r0 = jnp.sum(jnp.where(lane == 0, meta, 0.0), axis=1, keepdims=True)
    r1 = jnp.sum(jnp.where(lane == 1, meta, 0.0), axis=1, keepdims=True)
    e0 = jnp.sum(jnp.where(lane == 2, meta, 0.0), axis=1, keepdims=True)
    e1 = jnp.sum(jnp.where(lane == 3, meta, 0.0), axis=1, keepdims=True)
    pos0 = r0
    pos1 = r1
    bexp = jnp.full((1, LANES), -1.0)
    bnr = jnp.zeros((1, LANES))
    bidx = lane1.astype(jnp.float32)
    for e in range(N_EXPERTS):
        one_e = jnp.where(lane1 == e, 1.0, 0.0)
        base_e = jnp.sum(one_e * base)
        cnt_e = jnp.sum(one_e * cnt)
        bs_e = base_e * (1.0 / M_BLK)
        pos0 = pos0 + jnp.where(e0 == e, base_e, 0.0)
        pos1 = pos1 + jnp.where(e1 == e, base_e, 0.0)
        bexp = bexp + jnp.where(bidx >= bs_e, 1.0, 0.0)
        nr_e = jnp.clip(cnt_e - (bidx - bs_e) * M_BLK, 0.0, float(M_BLK))
        bnr = bnr + jnp.where(bidx >= bs_e, nr_e - bnr, 0.0)
    pos0_ref[...] = pos0.astype(jnp.int32)
    pos1_ref[...] = pos1.astype(jnp.int32)
    bexp_ref[...] = jnp.clip(bexp, 0.0, N_EXPERTS - 1.0).astype(jnp.int32)
    bnr_ref[...] = bnr.astype(jnp.int32)


def _sc_scatter(xn, pos0, pos1):
    mesh = plsc.VectorSubcoreMesh(core_axis_name="c", subcore_axis_name="s")
    n_tok = xn.shape[0]
    per_w = n_tok // N_WORKERS

    def body(xn_hbm, p0_hbm, p1_hbm, buf_hbm, idx0_v, idx1_v, rows_v,
             sem0, sem1):
        wid = lax.axis_index("s") * NC + lax.axis_index("c")
        for k in range(per_w // SC_CHUNK):
            start = wid * per_w + k * SC_CHUNK
            pltpu.sync_copy(p0_hbm.at[pl.ds(start, SC_CHUNK)], idx0_v)
            pltpu.sync_copy(p1_hbm.at[pl.ds(start, SC_CHUNK)], idx1_v)
            pltpu.sync_copy(xn_hbm.at[pl.ds(start, SC_CHUNK)], rows_v)
            c0 = pltpu.async_copy(rows_v, buf_hbm.at[idx0_v], sem0)
            c1 = pltpu.async_copy(rows_v, buf_hbm.at[idx1_v], sem1)
            c0.wait()
            c1.wait()

    return pl.kernel(
        body,
        out_type=jax.ShapeDtypeStruct((BUF_ROWS, D_HALF), jnp.int32),
        mesh=mesh,
        scratch_types=[
            pltpu.VMEM((SC_CHUNK,), jnp.int32),
            pltpu.VMEM((SC_CHUNK,), jnp.int32),
            pltpu.VMEM((SC_CHUNK, D_HALF), jnp.int32),
            pltpu.SemaphoreType.DMA,
            pltpu.SemaphoreType.DMA,
        ],
    )(xn, pos0, pos1)


def _sc_gather(y, pos0, pos1):
    mesh = plsc.VectorSubcoreMesh(core_axis_name="c", subcore_axis_name="s")
    n_tok = pos0.shape[0]
    per_w = n_tok // N_WORKERS

    def body(y_hbm, p0_hbm, p1_hbm, y0_hbm, y1_hbm, idx_v, rows_v, sem):
        wid = lax.axis_index("s") * NC + lax.axis_index("c")
        for k in range(per_w // SC_CHUNK):
            start = wid * per_w + k * SC_CHUNK
            pltpu.sync_copy(p0_hbm.at[pl.ds(start, SC_CHUNK)], idx_v)
            pltpu.async_copy(y_hbm.at[idx_v], rows_v, sem).wait()
            pltpu.sync_copy(rows_v, y0_hbm.at[pl.ds(start, SC_CHUNK)])
            pltpu.sync_copy(p1_hbm.at[pl.ds(start, SC_CHUNK)], idx_v)
            pltpu.async_copy(y_hbm.at[idx_v], rows_v, sem).wait()
            pltpu.sync_copy(rows_v, y1_hbm.at[pl.ds(start, SC_CHUNK)])

    return pl.kernel(
        body,
        out_type=[
            jax.ShapeDtypeStruct((n_tok, D_HALF), jnp.int32),
            jax.ShapeDtypeStruct((n_tok, D_HALF), jnp.int32),
        ],
        mesh=mesh,
        scratch_types=[
            pltpu.VMEM((SC_CHUNK,), jnp.int32),
            pltpu.VMEM((SC_CHUNK, D_HALF), jnp.int32),
            pltpu.SemaphoreType.DMA,
        ],
    )(y, pos0, pos1)


def _stage_b(bexp_ref, bnr_ref, buf_ref, wg_ref, wu_ref, wd_ref, y_ref):
    b = pl.program_id(0)

    @pl.when(bnr_ref[b] > 0)
    def _compute():
        xb = _unpack_rows(buf_ref[...]).astype(jnp.bfloat16)
        acc = jnp.zeros((M_BLK, D_MODEL), jnp.float32)
        for h in range(0, D_EXPERT, H_CHUNK):
            g = jnp.dot(xb, wg_ref[0, :, h:h + H_CHUNK],
                        preferred_element_type=jnp.float32)
            u = jnp.dot(xb, wu_ref[0, :, h:h + H_CHUNK],
                        preferred_element_type=jnp.float32)
            hid = (g * jax.lax.logistic(g) * u).astype(jnp.bfloat16)
            acc = acc + jnp.dot(hid, wd_ref[0, h:h + H_CHUNK, :],
                                preferred_element_type=jnp.float32)
        y_ref[...] = _pack_rows(acc)


def _stage_c(x_ref, y0_ref, y1_ref, meta_ref, out_ref):
    meta = meta_ref[...]
    lane = jax.lax.broadcasted_iota(jnp.int32, meta.shape, 1)
    p0 = jnp.sum(jnp.where(lane == 4, meta, 0.0), axis=1, keepdims=True)
    p1 = jnp.sum(jnp.where(lane == 5, meta, 0.0), axis=1, keepdims=True)
    y0 = _unpack_rows(y0_ref[...])
    y1 = _unpack_rows(y1_ref[...])
    out_ref[...] = x_ref[...] + p0 * y0 + p1 * y1


def kernel(x, norm_w, router_w, Wg, Wu, Wd):
    b, s, d = x.shape
    n_tok = b * s
    x_flat = x.reshape(n_tok, d)
    nw = norm_w.reshape(1, d)
    rw_pad = jnp.zeros((d, LANES), jnp.float32).at[:, :N_EXPERTS].set(router_w)
    rr = jax.lax.broadcasted_iota(jnp.int32, (TB, TB), 0)
    cc = jax.lax.broadcasted_iota(jnp.int32, (TB, TB), 1)
    ltri = jnp.where(rr > cc, 1.0, 0.0).astype(jnp.bfloat16)

    n_tb = n_tok // TB
    xn, meta, cnt = pl.pallas_call(
        _stage_a,
        grid=(n_tb,),
        in_specs=[
            pl.BlockSpec((TB, d), lambda i: (i, 0)),
            pl.BlockSpec((1, d), lambda i: (0, 0)),
            pl.BlockSpec((d, LANES), lambda i: (0, 0)),
            pl.BlockSpec((TB, TB), lambda i: (0, 0)),
        ],
        out_specs=[
            pl.BlockSpec((TB, D_HALF), lambda i: (i, 0)),
            pl.BlockSpec((TB, LANES), lambda i: (i, 0)),
            pl.BlockSpec((1, LANES), lambda i: (0, 0)),
        ],
        out_shape=[
            jax.ShapeDtypeStruct((n_tok, D_HALF), jnp.int32),
            jax.ShapeDtypeStruct((n_tok, LANES), jnp.float32),
            jax.ShapeDtypeStruct((1, LANES), jnp.float32),
        ],
        scratch_shapes=[pltpu.VMEM((1, LANES), jnp.float32)],
        compiler_params=pltpu.CompilerParams(
            dimension_semantics=("arbitrary",)),
    )(x_flat, nw, rw_pad, ltri)

    pos0, pos1, bexp, bnr = pl.pallas_call(
        _stage_a3,
        grid=(n_tb,),
        in_specs=[
            pl.BlockSpec((1, LANES), lambda i: (0, 0)),
            pl.BlockSpec((TB, LANES), lambda i: (i, 0)),
        ],
        out_specs=[
            pl.BlockSpec((TB, 1), lambda i: (i, 0)),
            pl.BlockSpec((TB, 1), lambda i: (i, 0)),
            pl.BlockSpec((1, LANES), lambda i: (0, 0)),
            pl.BlockSpec((1, LANES), lambda i: (0, 0)),
        ],
        out_shape=[
            jax.ShapeDtypeStruct((n_tok, 1), jnp.int32),
            jax.ShapeDtypeStruct((n_tok, 1), jnp.int32),
            jax.ShapeDtypeStruct((1, LANES), jnp.int32),
            jax.ShapeDtypeStruct((1, LANES), jnp.int32),
        ],
        compiler_params=pltpu.CompilerParams(
            dimension_semantics=("arbitrary",)),
    )(cnt, meta)

    pos0_f = pos0.reshape(n_tok)
    pos1_f = pos1.reshape(n_tok)
    buf = _sc_scatter(xn, pos0_f, pos1_f)

    bexp_s = bexp.reshape(LANES)[:N_BLK]
    bnr_s = bnr.reshape(LANES)[:N_BLK]
    wg16 = Wg.astype(jnp.bfloat16)
    wu16 = Wu.astype(jnp.bfloat16)
    wd16 = Wd.astype(jnp.bfloat16)

    y = pl.pallas_call(
        _stage_b,
        grid_spec=pltpu.PrefetchScalarGridSpec(
            num_scalar_prefetch=2,
            grid=(N_BLK,),
            in_specs=[
                pl.BlockSpec((M_BLK, D_HALF), lambda b, be, bn: (b, 0)),
                pl.BlockSpec((1, d, D_EXPERT), lambda b, be, bn: (be[b], 0, 0)),
                pl.BlockSpec((1, d, D_EXPERT), lambda b, be, bn: (be[b], 0, 0)),
                pl.BlockSpec((1, D_EXPERT, d), lambda b, be, bn: (be[b], 0, 0)),
            ],
            out_specs=pl.BlockSpec((M_BLK, D_HALF),
                                   lambda b, be, bn: (b, 0)),
        ),
        out_shape=jax.ShapeDtypeStruct((BUF_ROWS, D_HALF), jnp.int32),
        compiler_params=pltpu.CompilerParams(
            dimension_semantics=("arbitrary",)),
    )(bexp_s, bnr_s, buf, wg16, wu16, wd16)

    y0, y1 = _sc_gather(y, pos0_f, pos1_f)

    out = pl.pallas_call(
        _stage_c,
        grid=(n_tb,),
        in_specs=[
            pl.BlockSpec((TB, d), lambda i: (i, 0)),
            pl.BlockSpec((TB, D_HALF), lambda i: (i, 0)),
            pl.BlockSpec((TB, D_HALF), lambda i: (i, 0)),
            pl.BlockSpec((TB, LANES), lambda i: (i, 0)),
        ],
        out_specs=pl.BlockSpec((TB, d), lambda i: (i, 0)),
        out_shape=jax.ShapeDtypeStruct((n_tok, d), jnp.float32),
    )(x_flat, y0, y1, meta)

    return out.reshape(b, s, d)
